# initial kernel scaffold (unmeasured)
import jax
import jax.numpy as jnp
from jax import lax
from jax.experimental import pallas as pl
from jax.experimental.pallas import tpu as pltpu

N_DEV = 4
TAPS = 4
HALO = TAPS - 1


def kernel(x, k):
    b, s, c = x.shape

    def body(x_ref, k_ref, out_ref, halo_ref, send_sem, recv_sem):
        my = lax.axis_index("i")
        right = lax.rem(my + 1, N_DEV)

        rdma = pltpu.make_async_remote_copy(
            src_ref=x_ref.at[:, pl.ds(s - HALO, HALO), :],
            dst_ref=halo_ref,
            send_sem=send_sem,
            recv_sem=recv_sem,
            device_id=(right,),
            device_id_type=pl.DeviceIdType.MESH,
        )
        rdma.start()
        rdma.wait()

        @pl.when(my == 0)
        def _():
            halo_ref[...] = jnp.zeros_like(halo_ref)

        kv = k_ref[...]
        for bb in range(b):
            xb = x_ref[bb, :, :]
            hb = halo_ref[bb, :, :]
            acc = xb * kv[TAPS - 1][None, :]
            for t in range(TAPS - 1):
                sh = HALO - t
                shifted = jnp.concatenate(
                    [hb[HALO - sh :, :], xb[: s - sh, :]], axis=0
                )
                acc = acc + shifted * kv[t][None, :]
            out_ref[bb, :, :] = (acc * jax.nn.sigmoid(acc)).astype(out_ref.dtype)

    out_shape = jax.ShapeDtypeStruct((b, s, c), jnp.bfloat16)
    return pl.pallas_call(
        body,
        out_shape=out_shape,
        in_specs=[
            pl.BlockSpec(memory_space=pltpu.VMEM),
            pl.BlockSpec(memory_space=pltpu.VMEM),
        ],
        out_specs=pl.BlockSpec(memory_space=pltpu.VMEM),
        scratch_shapes=[
            pltpu.VMEM((b, HALO, c), x.dtype),
            pltpu.SemaphoreType.DMA,
            pltpu.SemaphoreType.DMA,
        ],
        compiler_params=pltpu.CompilerParams(collective_id=0),
    )(x, k)


# baseline (device time: 60619 ns/iter reference)
import jax
import jax.numpy as jnp
from jax import lax
from jax.experimental import pallas as pl
from jax.experimental.pallas import tpu as pltpu

N_DEV = 4
TAPS = 4
HALO = TAPS - 1


def kernel(x, k):
    b, s, c = x.shape

    def body(x_ref, k_ref, out_ref, halo_ref, send_sem, recv_sem):
        my = lax.axis_index("i")
        right = lax.rem(my + 1, N_DEV)

        rdma = pltpu.make_async_remote_copy(
            src_ref=x_ref.at[:, pl.ds(s - HALO, HALO), :],
            dst_ref=halo_ref,
            send_sem=send_sem,
            recv_sem=recv_sem,
            device_id=(right,),
            device_id_type=pl.DeviceIdType.MESH,
        )
        rdma.start()
        rdma.wait()

        @pl.when(my == 0)
        def _():
            halo_ref[...] = jnp.zeros_like(halo_ref)

        kv = k_ref[...]
        for bb in range(b):
            xb = x_ref[bb, :, :]
            hb = halo_ref[bb, :, :]
            acc = xb * kv[TAPS - 1][None, :]
            for t in range(TAPS - 1):
                sh = HALO - t
                shifted = jnp.concatenate(
                    [hb[HALO - sh :, :], xb[: s - sh, :]], axis=0
                )
                acc = acc + shifted * kv[t][None, :]
            out_ref[bb, :, :] = (acc * jax.nn.sigmoid(acc)).astype(out_ref.dtype)

    out_shape = jax.ShapeDtypeStruct((b, s, c), jnp.bfloat16)
    return pl.pallas_call(
        body,
        out_shape=out_shape,
        in_specs=[
            pl.BlockSpec(memory_space=pltpu.VMEM),
            pl.BlockSpec(memory_space=pltpu.VMEM),
        ],
        out_specs=pl.BlockSpec(memory_space=pltpu.VMEM),
        scratch_shapes=[
            pltpu.VMEM((b, HALO, c), x.dtype),
            pltpu.SemaphoreType.DMA,
            pltpu.SemaphoreType.DMA,
        ],
        compiler_params=pltpu.CompilerParams(
            vmem_limit_bytes=100 * 1024 * 1024
        ),
    )(x, k)


# device time: 40494 ns/iter; 1.4970x vs baseline; 1.4970x over previous
import jax
import jax.numpy as jnp
from jax import lax
from jax.experimental import pallas as pl
from jax.experimental.pallas import tpu as pltpu

N_DEV = 4
TAPS = 4
HALO = TAPS - 1
CHUNK = 512


def kernel(x, k):
    b, s, c = x.shape
    nchunk = s // CHUNK

    def body(x_ref, k_ref, x_any, out_ref, halo_ref, tail_ref, send_sem, recv_sem):
        my = lax.axis_index("i")
        right = lax.rem(my + 1, N_DEV)
        bb = pl.program_id(0)
        j = pl.program_id(1)

        @pl.when((bb == 0) & (j == 0))
        def _():
            rdma = pltpu.make_async_remote_copy(
                src_ref=x_any.at[:, pl.ds(s - HALO, HALO), :],
                dst_ref=halo_ref,
                send_sem=send_sem,
                recv_sem=recv_sem,
                device_id=(right,),
                device_id_type=pl.DeviceIdType.MESH,
            )
            rdma.start()
            rdma.wait()

            @pl.when(my == 0)
            def _():
                halo_ref[...] = jnp.zeros_like(halo_ref)

        xb = x_ref[0]
        prev = jnp.where(
            (j == 0)[None, None],
            halo_ref[bb],
            tail_ref[...],
        )
        kv = k_ref[...]
        acc = xb * kv[TAPS - 1][None, :]
        for t in range(TAPS - 1):
            sh = HALO - t
            shifted = jnp.concatenate(
                [prev[HALO - sh :, :], xb[: CHUNK - sh, :]], axis=0
            )
            acc = acc + shifted * kv[t][None, :]
        out_ref[0] = (acc * jax.nn.sigmoid(acc)).astype(out_ref.dtype)
        tail_ref[...] = xb[CHUNK - HALO :, :]

    out_shape = jax.ShapeDtypeStruct((b, s, c), jnp.bfloat16)
    return pl.pallas_call(
        body,
        grid=(b, nchunk),
        out_shape=out_shape,
        in_specs=[
            pl.BlockSpec((1, CHUNK, c), lambda bb, j: (bb, j, 0)),
            pl.BlockSpec((TAPS, c), lambda bb, j: (0, 0)),
            pl.BlockSpec(memory_space=pl.ANY),
        ],
        out_specs=pl.BlockSpec((1, CHUNK, c), lambda bb, j: (bb, j, 0)),
        scratch_shapes=[
            pltpu.VMEM((b, HALO, c), x.dtype),
            pltpu.VMEM((HALO, c), x.dtype),
            pltpu.SemaphoreType.DMA,
            pltpu.SemaphoreType.DMA,
        ],
        compiler_params=pltpu.CompilerParams(
            dimension_semantics=("arbitrary", "arbitrary"),
        ),
    )(x, k, x)


# device time: 36580 ns/iter; 1.6572x vs baseline; 1.1070x over previous
import jax
import jax.numpy as jnp
from jax import lax
from jax.experimental import pallas as pl
from jax.experimental.pallas import tpu as pltpu

N_DEV = 4
TAPS = 4
HALO = TAPS - 1
CHUNK = 512


def kernel(x, k):
    b, s, c = x.shape
    nchunk = s // CHUNK

    def body(x_ref, k_ref, x_any, out_ref, halo_ref, tail_ref, send_sem, recv_sem):
        my = lax.axis_index("i")
        right = lax.rem(my + 1, N_DEV)
        bb = pl.program_id(0)
        j = pl.program_id(1)

        @pl.when((bb == 0) & (j == 0))
        def _():
            rdma = pltpu.make_async_remote_copy(
                src_ref=x_any.at[:, pl.ds(s - HALO, HALO), :],
                dst_ref=halo_ref,
                send_sem=send_sem,
                recv_sem=recv_sem,
                device_id=(right,),
                device_id_type=pl.DeviceIdType.MESH,
            )
            rdma.start()
            rdma.wait()

            @pl.when(my == 0)
            def _():
                halo_ref[...] = jnp.zeros_like(halo_ref)

        xb = x_ref[0].astype(jnp.bfloat16)
        prev = jnp.where(
            (j == 0)[None, None],
            halo_ref[bb],
            tail_ref[...],
        ).astype(jnp.bfloat16)
        kv = k_ref[...].astype(jnp.bfloat16)
        acc = xb * kv[TAPS - 1][None, :]
        for t in range(TAPS - 1):
            sh = HALO - t
            shifted = jnp.concatenate(
                [prev[HALO - sh :, :], xb[: CHUNK - sh, :]], axis=0
            )
            acc = acc + shifted * kv[t][None, :]
        out_ref[0] = acc * jax.nn.sigmoid(acc)
        tail_ref[...] = x_ref[0, CHUNK - HALO :, :]

    out_shape = jax.ShapeDtypeStruct((b, s, c), jnp.bfloat16)
    return pl.pallas_call(
        body,
        grid=(b, nchunk),
        out_shape=out_shape,
        in_specs=[
            pl.BlockSpec((1, CHUNK, c), lambda bb, j: (bb, j, 0)),
            pl.BlockSpec((TAPS, c), lambda bb, j: (0, 0)),
            pl.BlockSpec(memory_space=pl.ANY),
        ],
        out_specs=pl.BlockSpec((1, CHUNK, c), lambda bb, j: (bb, j, 0)),
        scratch_shapes=[
            pltpu.VMEM((b, HALO, c), x.dtype),
            pltpu.VMEM((HALO, c), x.dtype),
            pltpu.SemaphoreType.DMA,
            pltpu.SemaphoreType.DMA,
        ],
        compiler_params=pltpu.CompilerParams(
            dimension_semantics=("arbitrary", "arbitrary"),
        ),
    )(x, k, x)


# device time: 27704 ns/iter; 2.1881x vs baseline; 1.3204x over previous
import jax
import jax.numpy as jnp
from jax import lax
from jax.experimental import pallas as pl
from jax.experimental.pallas import tpu as pltpu

N_DEV = 4
TAPS = 4
HALO = TAPS - 1
CHUNK = 2048


def _halo_exchange(x):
    b, s, c = x.shape

    def body(x_any, halo_ref, send_sem, recv_sem):
        my = lax.axis_index("i")
        right = lax.rem(my + 1, N_DEV)
        barrier_sem = pltpu.get_barrier_semaphore()
        pl.semaphore_signal(
            barrier_sem, inc=1,
            device_id=(right,), device_id_type=pl.DeviceIdType.MESH,
        )
        pl.semaphore_wait(barrier_sem, 1)
        rdma = pltpu.make_async_remote_copy(
            src_ref=x_any.at[:, pl.ds(s - HALO, HALO), :],
            dst_ref=halo_ref,
            send_sem=send_sem,
            recv_sem=recv_sem,
            device_id=(right,),
            device_id_type=pl.DeviceIdType.MESH,
        )
        rdma.start()
        rdma.wait()

        @pl.when(my == 0)
        def _():
            halo_ref[...] = jnp.zeros_like(halo_ref)

    return pl.pallas_call(
        body,
        out_shape=jax.ShapeDtypeStruct((b, HALO, c), x.dtype),
        in_specs=[pl.BlockSpec(memory_space=pl.ANY)],
        out_specs=pl.BlockSpec(memory_space=pltpu.VMEM),
        scratch_shapes=[
            pltpu.SemaphoreType.DMA,
            pltpu.SemaphoreType.DMA,
        ],
        compiler_params=pltpu.CompilerParams(collective_id=0),
    )(x)


def kernel(x, k):
    b, s, c = x.shape
    nchunk = s // CHUNK
    halo = _halo_exchange(x)

    def body(x_ref, k_ref, halo_in, out_ref, tail_ref):
        bb = pl.program_id(0)
        j = pl.program_id(1)

        xb = x_ref[0].astype(jnp.bfloat16)

        @pl.when(j == 0)
        def _():
            tail_ref[...] = halo_in[bb]

        prev = tail_ref[...].astype(jnp.bfloat16)
        kv = k_ref[...].astype(jnp.bfloat16)
        acc = xb * kv[TAPS - 1][None, :]
        for t in range(TAPS - 1):
            sh = HALO - t
            shifted = jnp.concatenate(
                [prev[HALO - sh :, :], xb[: CHUNK - sh, :]], axis=0
            )
            acc = acc + shifted * kv[t][None, :]
        out_ref[0] = acc * jax.nn.sigmoid(acc)
        tail_ref[...] = x_ref[0, CHUNK - HALO :, :]

    out_shape = jax.ShapeDtypeStruct((b, s, c), jnp.bfloat16)
    return pl.pallas_call(
        body,
        grid=(b, nchunk),
        out_shape=out_shape,
        in_specs=[
            pl.BlockSpec((1, CHUNK, c), lambda bb, j: (bb, j, 0)),
            pl.BlockSpec((TAPS, c), lambda bb, j: (0, 0)),
            pl.BlockSpec((b, HALO, c), lambda bb, j: (0, 0, 0)),
        ],
        out_specs=pl.BlockSpec((1, CHUNK, c), lambda bb, j: (bb, j, 0)),
        scratch_shapes=[
            pltpu.VMEM((HALO, c), x.dtype),
        ],
        compiler_params=pltpu.CompilerParams(
            dimension_semantics=("arbitrary", "arbitrary"),
        ),
    )(x, k, halo)


# device time: 26667 ns/iter; 2.2732x vs baseline; 1.0389x over previous
import jax
import jax.numpy as jnp
from jax import lax
from jax.experimental import pallas as pl
from jax.experimental.pallas import tpu as pltpu

N_DEV = 4
TAPS = 4
HALO = TAPS - 1
CHUNK = 2048


def _halo_exchange(x):
    b, s, c = x.shape

    def body(x_any, halo_ref, send_sem, recv_sem):
        my = lax.axis_index("i")
        right = lax.rem(my + 1, N_DEV)
        barrier_sem = pltpu.get_barrier_semaphore()
        pl.semaphore_signal(barrier_sem, inc=1)
        pl.semaphore_wait(barrier_sem, 1)
        rdma = pltpu.make_async_remote_copy(
            src_ref=x_any.at[:, pl.ds(s - HALO, HALO), :],
            dst_ref=halo_ref,
            send_sem=send_sem,
            recv_sem=recv_sem,
            device_id=(right,),
            device_id_type=pl.DeviceIdType.MESH,
        )
        rdma.start()
        rdma.wait()

        @pl.when(my == 0)
        def _():
            halo_ref[...] = jnp.zeros_like(halo_ref)

    return pl.pallas_call(
        body,
        out_shape=jax.ShapeDtypeStruct((b, HALO, c), x.dtype),
        in_specs=[pl.BlockSpec(memory_space=pl.ANY)],
        out_specs=pl.BlockSpec(memory_space=pltpu.VMEM),
        scratch_shapes=[
            pltpu.SemaphoreType.DMA,
            pltpu.SemaphoreType.DMA,
        ],
        compiler_params=pltpu.CompilerParams(collective_id=0),
    )(x)


def kernel(x, k):
    b, s, c = x.shape
    nchunk = s // CHUNK
    halo = _halo_exchange(x)

    def body(x_ref, k_ref, halo_in, out_ref, tail_ref):
        bb = pl.program_id(0)
        j = pl.program_id(1)

        xb = x_ref[0].astype(jnp.bfloat16)

        if nchunk == 1:
            prev = halo_in[bb].astype(jnp.bfloat16)
        else:
            @pl.when(j == 0)
            def _():
                tail_ref[...] = halo_in[bb]

            prev = tail_ref[...].astype(jnp.bfloat16)
        kv = k_ref[...].astype(jnp.bfloat16)
        acc = xb * kv[TAPS - 1][None, :]
        for t in range(TAPS - 1):
            sh = HALO - t
            shifted = jnp.concatenate(
                [prev[HALO - sh :, :], xb[: CHUNK - sh, :]], axis=0
            )
            acc = acc + shifted * kv[t][None, :]
        out_ref[0] = acc * jax.nn.sigmoid(acc)
        if nchunk > 1:
            tail_ref[...] = x_ref[0, CHUNK - HALO :, :]

    out_shape = jax.ShapeDtypeStruct((b, s, c), jnp.bfloat16)
    return pl.pallas_call(
        body,
        grid=(b, nchunk),
        out_shape=out_shape,
        in_specs=[
            pl.BlockSpec((1, CHUNK, c), lambda bb, j: (bb, j, 0)),
            pl.BlockSpec((TAPS, c), lambda bb, j: (0, 0)),
            pl.BlockSpec((b, HALO, c), lambda bb, j: (0, 0, 0)),
        ],
        out_specs=pl.BlockSpec((1, CHUNK, c), lambda bb, j: (bb, j, 0)),
        scratch_shapes=[
            pltpu.VMEM((HALO, c), x.dtype),
        ],
        compiler_params=pltpu.CompilerParams(
            dimension_semantics=("arbitrary", "arbitrary"),
        ),
    )(x, k, halo)
